# baseline (speedup 1.0000x reference)
import functools

import jax
import jax.numpy as jnp
from jax import lax
from jax.experimental import pallas as pl
from jax.experimental.pallas import tpu as pltpu
from jax.experimental.pallas import tpu_sc as plsc


@functools.lru_cache(maxsize=None)
def _make_gather(B, D, NC, NS):
    NW = NC * NS
    assert B % (8 * NW) == 0
    b_per_w = B // NW
    L = 16
    mesh = plsc.VectorSubcoreMesh(core_axis_name="c", subcore_axis_name="s")

    @functools.partial(
        pl.kernel,
        mesh=mesh,
        out_type=jax.ShapeDtypeStruct((B, D), jnp.float32),
        scratch_types=[
            pltpu.VMEM((b_per_w,), jnp.int32),
            pltpu.VMEM((b_per_w // 2, D // 2), jnp.int32),
            pltpu.VMEM((b_per_w // 2, D // 2), jnp.int32),
            pltpu.VMEM((b_per_w // 2, D), jnp.float32),
            [pltpu.SemaphoreType.DMA] * 2,
        ],
        compiler_params=pltpu.CompilerParams(needs_layout_passes=False),
    )
    def k(table_hbm, idx_hbm, out_hbm, idx_v, rows_h0, rows_h1, rows_v, sems):
        wid = lax.axis_index("s") * NC + lax.axis_index("c")
        base = wid * b_per_w
        pltpu.sync_copy(idx_hbm.at[pl.ds(base, b_per_w)], idx_v)

        half = b_per_w // 2
        hj = half // L
        even = lax.iota(jnp.int32, L) * 2

        bufs = (rows_h0, rows_h1)
        for ph in range(2):

            @plsc.parallel_loop(0, hj)
            def issue(j, ph=ph):
                v = idx_v[pl.ds((ph * hj + j) * L, L)]
                for t in range(L):
                    pltpu.async_copy(
                        table_hbm.at[pl.ds(v[t], 1)],
                        bufs[ph].at[pl.ds(j * L + t, 1)],
                        sems[ph],
                    )

        for ph in range(2):
            pltpu.make_async_copy(
                table_hbm.at[pl.ds(0, half)], bufs[ph], sems[ph]
            ).wait()

            @plsc.parallel_loop(0, half)
            def widen(i, ph=ph):
                ii = jnp.full((L,), i, jnp.int32)
                for c in range(D // (2 * L)):
                    w = bufs[ph][i, pl.ds(c * L, L)]
                    lo = plsc.bitcast(
                        lax.shift_left(w, jnp.full((L,), 16, jnp.int32)),
                        jnp.float32,
                    )
                    hi = plsc.bitcast(
                        w & jnp.full((L,), -65536, jnp.int32), jnp.float32
                    )
                    pos = even + (c * 2 * L)
                    plsc.store_scatter(rows_v, [ii, pos], lo)
                    plsc.store_scatter(rows_v, [ii, pos + 1], hi)

            pltpu.sync_copy(
                rows_v, out_hbm.at[pl.ds(base + ph * half, half)]
            )

    return k


def kernel(style_id, embed_weight):
    V, D = embed_weight.shape
    (B,) = style_id.shape
    info = plsc.get_sparse_core_info()
    idx = style_id.astype(jnp.int32)
    tab_i32 = jax.lax.bitcast_convert_type(
        embed_weight.astype(jnp.bfloat16).reshape(V, D // 2, 2), jnp.int32
    )
    return _make_gather(B, D, info.num_cores, info.num_subcores)(
        tab_i32, idx
    )


# speedup vs baseline: 3.9917x; 3.9917x over previous
import functools

import jax
import jax.numpy as jnp
from jax import lax
from jax.experimental import pallas as pl
from jax.experimental.pallas import tpu as pltpu
from jax.experimental.pallas import tpu_sc as plsc


@functools.lru_cache(maxsize=None)
def _make_gather(B, D, NC, NS):
    NW = NC * NS
    assert B % (8 * NW) == 0
    b_per_w = B // NW
    L = 16
    mesh = plsc.VectorSubcoreMesh(core_axis_name="c", subcore_axis_name="s")

    @functools.partial(
        pl.kernel,
        mesh=mesh,
        out_type=jax.ShapeDtypeStruct((B, D), jnp.float32),
        scratch_types=[
            pltpu.VMEM((b_per_w,), jnp.int32),
            pltpu.VMEM((b_per_w, D), jnp.float32),
            pltpu.SemaphoreType.DMA,
        ],
    )
    def k(table_hbm, idx_hbm, out_hbm, idx_v, rows_v, sem):
        wid = lax.axis_index("s") * NC + lax.axis_index("c")
        base = wid * b_per_w
        pltpu.sync_copy(idx_hbm.at[pl.ds(base, b_per_w)], idx_v)

        def issue(j, _):
            v = idx_v[pl.ds(j * L, L)]
            for t in range(L):
                pltpu.async_copy(
                    table_hbm.at[pl.ds(v[t], 1)],
                    rows_v.at[pl.ds(j * L + t, 1)],
                    sem,
                )
            return _

        lax.fori_loop(0, b_per_w // L, issue, None)
        pltpu.make_async_copy(
            table_hbm.at[pl.ds(0, b_per_w)], rows_v, sem
        ).wait()
        pltpu.sync_copy(rows_v, out_hbm.at[pl.ds(base, b_per_w)])

    return k


def kernel(style_id, embed_weight):
    V, D = embed_weight.shape
    (B,) = style_id.shape
    info = plsc.get_sparse_core_info()
    idx = style_id.astype(jnp.int32)
    return _make_gather(B, D, info.num_cores, info.num_subcores)(
        embed_weight, idx
    )


# speedup vs baseline: 4.0173x; 1.0064x over previous
import functools

import jax
import jax.numpy as jnp
from jax import lax
from jax.experimental import pallas as pl
from jax.experimental.pallas import tpu as pltpu
from jax.experimental.pallas import tpu_sc as plsc


@functools.lru_cache(maxsize=None)
def _make_gather(B, D, NC, NS):
    NW = NC * NS
    assert B % (8 * NW) == 0
    b_per_w = B // NW
    L = 16
    mesh = plsc.VectorSubcoreMesh(core_axis_name="c", subcore_axis_name="s")

    @functools.partial(
        pl.kernel,
        mesh=mesh,
        out_type=jax.ShapeDtypeStruct((B, D), jnp.float32),
        scratch_types=[
            pltpu.VMEM((b_per_w,), jnp.int32),
            pltpu.VMEM((b_per_w, D), jnp.float32),
            pltpu.SemaphoreType.DMA,
        ],
    )
    def k(table_hbm, idx_hbm, out_hbm, idx_v, rows_v, sem):
        wid = lax.axis_index("s") * NC + lax.axis_index("c")
        base = wid * b_per_w
        pltpu.sync_copy(idx_hbm.at[pl.ds(base, b_per_w)], idx_v)

        @plsc.parallel_loop(0, b_per_w // L)
        def issue(j):
            v = idx_v[pl.ds(j * L, L)]
            for t in range(L):
                pltpu.async_copy(
                    table_hbm.at[pl.ds(v[t], 1)],
                    rows_v.at[pl.ds(j * L + t, 1)],
                    sem,
                )
        pltpu.make_async_copy(
            table_hbm.at[pl.ds(0, b_per_w)], rows_v, sem
        ).wait()
        pltpu.sync_copy(rows_v, out_hbm.at[pl.ds(base, b_per_w)])

    return k


def kernel(style_id, embed_weight):
    V, D = embed_weight.shape
    (B,) = style_id.shape
    info = plsc.get_sparse_core_info()
    idx = style_id.astype(jnp.int32)
    return _make_gather(B, D, info.num_cores, info.num_subcores)(
        embed_weight, idx
    )
